# Initial kernel scaffold; baseline (speedup 1.0000x reference)
import functools

import jax
import jax.numpy as jnp
from jax import lax
from jax.experimental import pallas as pl
from jax.experimental.pallas import tpu as pltpu
from jax.experimental.pallas import tpu_sc as plsc

_NC = 2
_NS = 16
_NW = _NC * _NS
_L = 16


def _sc_skipgram(b, c, k, dim, ch):
    ipw = b // _NW
    nchunk = ipw // ch
    nq = dim // _L

    mesh = plsc.VectorSubcoreMesh(
        core_axis_name="c", subcore_axis_name="s",
        num_cores=_NC, num_subcores=_NS)

    @functools.partial(
        pl.kernel,
        out_type=(
            jax.ShapeDtypeStruct((b, c), jnp.float32),
            jax.ShapeDtypeStruct((b, k), jnp.float32),
        ),
        mesh=mesh,
        scratch_types=[
            pltpu.VMEM((ch,), jnp.int32),
            pltpu.VMEM((ch * c,), jnp.int32),
            pltpu.VMEM((ch * k,), jnp.int32),
            pltpu.VMEM((ch, dim), jnp.float32),
            pltpu.VMEM((ch * c, dim), jnp.float32),
            pltpu.VMEM((ch * k, dim), jnp.float32),
            pltpu.VMEM((ch, c), jnp.float32),
            pltpu.VMEM((ch, k), jnp.float32),
            pltpu.SemaphoreType.DMA,
            pltpu.SemaphoreType.DMA,
            pltpu.SemaphoreType.DMA,
        ],
    )
    def kern(cw_hbm, cx_hbm, ng_hbm, ctab_hbm, wtab_hbm, pos_hbm, neg_hbm,
             idx_c, idx_x, idx_n, rows_c, rows_x, rows_n, out_p, out_n,
             sem0, sem1, sem2):
        wid = lax.axis_index("s") * _NC + lax.axis_index("c")

        def chunk(t, carry):
            gbase = wid * ipw + t * ch
            pltpu.sync_copy(cw_hbm.at[pl.ds(gbase, ch)], idx_c)
            pltpu.sync_copy(cx_hbm.at[pl.ds(gbase * c, ch * c)], idx_x)
            pltpu.sync_copy(ng_hbm.at[pl.ds(gbase * k, ch * k)], idx_n)
            d0 = pltpu.async_copy(ctab_hbm.at[idx_c], rows_c, sem0)
            d1 = pltpu.async_copy(wtab_hbm.at[idx_x], rows_x, sem1)
            d2 = pltpu.async_copy(wtab_hbm.at[idx_n], rows_n, sem2)
            d0.wait()
            d1.wait()
            d2.wait()

            def item(i, carry2):
                cvec = [rows_c[i, pl.ds(q * _L, _L)] for q in range(nq)]
                for j in range(c):
                    r = i * c + j
                    acc = cvec[0] * rows_x[r, pl.ds(0, _L)]
                    for q in range(1, nq):
                        acc = acc + cvec[q] * rows_x[r, pl.ds(q * _L, _L)]
                    out_p[i, j] = jnp.sum(acc)
                for j in range(k):
                    r = i * k + j
                    acc = cvec[0] * rows_n[r, pl.ds(0, _L)]
                    for q in range(1, nq):
                        acc = acc + cvec[q] * rows_n[r, pl.ds(q * _L, _L)]
                    out_n[i, j] = -jnp.sum(acc)
                return carry2

            lax.fori_loop(0, ch, item, 0)
            pltpu.sync_copy(out_p, pos_hbm.at[pl.ds(gbase, ch)])
            pltpu.sync_copy(out_n, neg_hbm.at[pl.ds(gbase, ch)])
            return carry

        lax.fori_loop(0, nchunk, chunk, 0)

    return kern


def kernel(center_word, context_words, negative_words, centerword_table,
           contextword_table):
    b, = center_word.shape
    c = context_words.shape[1]
    k = negative_words.shape[1]
    dim = centerword_table.shape[1]
    cw = center_word.astype(jnp.int32)
    cx = context_words.astype(jnp.int32).reshape(b * c)
    ng = negative_words.astype(jnp.int32).reshape(b * k)
    kern = _sc_skipgram(b, c, k, dim, ch=8)
    pos, neg = kern(cw, cx, ng, centerword_table, contextword_table)
    return (pos, neg)


# speedup vs baseline: 4.7714x; 4.7714x over previous
import functools

import jax
import jax.numpy as jnp
from jax import lax
from jax.experimental import pallas as pl
from jax.experimental.pallas import tpu as pltpu
from jax.experimental.pallas import tpu_sc as plsc

def _perm(x, p):
    dnums = lax.GatherDimensionNumbers(
        offset_dims=(), collapsed_slice_dims=(0,), start_index_map=(0,))
    return lax.gather(x, p[:, None], dimension_numbers=dnums,
                      slice_sizes=(1,),
                      mode=lax.GatherScatterMode.PROMISE_IN_BOUNDS)


_NC = 2
_NS = 16
_NW = _NC * _NS
_L = 16


def _sc_skipgram(b, c, k, dim, ch):
    ipw = b // _NW
    nchunk = ipw // ch
    nq = dim // _L
    cpad = -(-c // _L) * _L
    kpad = -(-k // _L) * _L

    mesh = plsc.VectorSubcoreMesh(
        core_axis_name="c", subcore_axis_name="s",
        num_cores=_NC, num_subcores=_NS)

    @functools.partial(
        pl.kernel,
        out_type=(
            jax.ShapeDtypeStruct((b, cpad), jnp.float32),
            jax.ShapeDtypeStruct((b, kpad), jnp.float32),
        ),
        mesh=mesh,
        compiler_params=pltpu.CompilerParams(use_tc_tiling_on_sc=False),
        scratch_types=[
            pltpu.VMEM((ch,), jnp.int32),
            pltpu.VMEM((ch * c,), jnp.int32),
            pltpu.VMEM((ch * k,), jnp.int32),
            pltpu.VMEM((ch, dim), jnp.float32),
            pltpu.VMEM((ch * c, dim), jnp.float32),
            pltpu.VMEM((ch * k, dim), jnp.float32),
            pltpu.VMEM((ch, cpad), jnp.float32),
            pltpu.VMEM((ch, kpad), jnp.float32),
            pltpu.SemaphoreType.DMA,
            pltpu.SemaphoreType.DMA,
            pltpu.SemaphoreType.DMA,
        ],
    )
    def kern(cw_hbm, cx_hbm, ng_hbm, ctab_hbm, wtab_hbm, pos_hbm, neg_hbm,
             idx_c, idx_x, idx_n, rows_c, rows_x, rows_n, out_p, out_n,
             sem0, sem1, sem2):
        wid = lax.axis_index("s") * _NC + lax.axis_index("c")

        def chunk(t, carry):
            gbase = wid * ipw + t * ch
            pltpu.sync_copy(cw_hbm.at[pl.ds(gbase, ch)], idx_c)
            pltpu.sync_copy(cx_hbm.at[pl.ds(gbase * c, ch * c)], idx_x)
            pltpu.sync_copy(ng_hbm.at[pl.ds(gbase * k, ch * k)], idx_n)
            d0 = pltpu.async_copy(ctab_hbm.at[idx_c], rows_c, sem0)
            d1 = pltpu.async_copy(wtab_hbm.at[idx_x], rows_x, sem1)
            d2 = pltpu.async_copy(wtab_hbm.at[idx_n], rows_n, sem2)
            d0.wait()
            d1.wait()
            d2.wait()

            lane = lax.iota(jnp.int32, _L)
            perms = [lax.iota(jnp.int32, _L) ^ d for d in (8, 4, 2, 1)]

            def item(i, carry2):
                cvec = [rows_c[i, pl.ds(q * _L, _L)] for q in range(nq)]

                def score(rows, r):
                    acc = cvec[0] * rows[r, pl.ds(0, _L)]
                    for q in range(1, nq):
                        acc = acc + cvec[q] * rows[r, pl.ds(q * _L, _L)]
                    for p in perms:
                        acc = acc + _perm(acc, p)
                    return acc

                for g in range(cpad // _L):
                    vec = jnp.zeros((_L,), jnp.float32)
                    for jj in range(min(_L, c - g * _L)):
                        s = score(rows_x, i * c + g * _L + jj)
                        vec = jnp.where(lane == jj, s, vec)
                    out_p[i, pl.ds(g * _L, _L)] = vec
                for g in range(kpad // _L):
                    vec = jnp.zeros((_L,), jnp.float32)
                    for jj in range(min(_L, k - g * _L)):
                        s = score(rows_n, i * k + g * _L + jj)
                        vec = jnp.where(lane == jj, s, vec)
                    out_n[i, pl.ds(g * _L, _L)] = -vec
                return carry2

            lax.fori_loop(0, ch, item, 0)
            pltpu.sync_copy(out_p, pos_hbm.at[pl.ds(gbase, ch)])
            pltpu.sync_copy(out_n, neg_hbm.at[pl.ds(gbase, ch)])
            return carry

        lax.fori_loop(0, nchunk, chunk, 0)

    return kern


def kernel(center_word, context_words, negative_words, centerword_table,
           contextword_table):
    b, = center_word.shape
    c = context_words.shape[1]
    k = negative_words.shape[1]
    dim = centerword_table.shape[1]
    cw = center_word.astype(jnp.int32)
    cx = context_words.astype(jnp.int32).reshape(b * c)
    ng = negative_words.astype(jnp.int32).reshape(b * k)
    kern = _sc_skipgram(b, c, k, dim, ch=8)
    pos, neg = kern(cw, cx, ng, centerword_table, contextword_table)
    return (pos[:, :c], neg[:, :k])
